# baseline (speedup 1.0000x reference)
import jax
import jax.numpy as jnp
from jax.experimental import pallas as pl

_N_ITERS = 12
_BLOCK_ROWS = 256


def _sparsemax_block(x_ref, o_ref):
    x = x_ref[...]
    n = x.shape[1]
    ones = jnp.ones((n, 1), x.dtype)
    mx = jnp.max(x, axis=1, keepdims=True)
    lo = mx - 1.0
    hi = mx

    def body(_, carry):
        lo, hi = carry
        mid = 0.5 * (lo + hi)
        f = jnp.dot(jnp.maximum(x - mid, 0.0), ones,
                    preferred_element_type=jnp.float32)
        gt = f > 1.0
        lo = jnp.where(gt, mid, lo)
        hi = jnp.where(gt, hi, mid)
        return lo, hi

    lo, hi = jax.lax.fori_loop(0, _N_ITERS, body, (lo, hi))
    mask = x > lo
    s = jnp.dot(jnp.where(mask, x, 0.0), ones,
                preferred_element_type=jnp.float32)
    k = jnp.dot(mask.astype(x.dtype), ones,
                preferred_element_type=jnp.float32)
    tau = jnp.clip((s - 1.0) / k, lo, hi)
    o_ref[...] = jnp.maximum(x - tau, 0.0)


def kernel(input):
    orig_shape = input.shape
    n = orig_shape[-1]
    x2 = input.reshape(-1, n)
    rows = x2.shape[0]
    out = pl.pallas_call(
        _sparsemax_block,
        grid=(rows // _BLOCK_ROWS,),
        in_specs=[pl.BlockSpec((_BLOCK_ROWS, n), lambda i: (i, 0))],
        out_specs=pl.BlockSpec((_BLOCK_ROWS, n), lambda i: (i, 0)),
        out_shape=jax.ShapeDtypeStruct((rows, n), x2.dtype),
    )(x2)
    return out.reshape(orig_shape)


# speedup vs baseline: 1.8440x; 1.8440x over previous
import jax
import jax.numpy as jnp
from jax.experimental import pallas as pl

_N_ITERS = 10
_BLOCK_ROWS = 256


def _sparsemax_block(x_ref, o_ref):
    x = x_ref[...]
    mx = jnp.max(x, axis=1, keepdims=True)
    lo = mx - 1.0
    hi = mx

    def body(_, carry):
        lo, hi = carry
        mid = 0.5 * (lo + hi)
        f = jnp.sum(jnp.maximum(x - mid, 0.0), axis=1, keepdims=True)
        gt = f > 1.0
        lo = jnp.where(gt, mid, lo)
        hi = jnp.where(gt, hi, mid)
        return lo, hi

    lo, hi = jax.lax.fori_loop(0, _N_ITERS, body, (lo, hi))
    mask = x > lo
    s = jnp.sum(jnp.where(mask, x, 0.0), axis=1, keepdims=True)
    k = jnp.sum(mask.astype(x.dtype), axis=1, keepdims=True)
    tau = jnp.clip((s - 1.0) / k, lo, hi)
    o_ref[...] = jnp.maximum(x - tau, 0.0)


def kernel(input):
    orig_shape = input.shape
    n = orig_shape[-1]
    x2 = input.reshape(-1, n)
    rows = x2.shape[0]
    out = pl.pallas_call(
        _sparsemax_block,
        grid=(rows // _BLOCK_ROWS,),
        in_specs=[pl.BlockSpec((_BLOCK_ROWS, n), lambda i: (i, 0))],
        out_specs=pl.BlockSpec((_BLOCK_ROWS, n), lambda i: (i, 0)),
        out_shape=jax.ShapeDtypeStruct((rows, n), x2.dtype),
    )(x2)
    return out.reshape(orig_shape)


# speedup vs baseline: 2.0055x; 1.0876x over previous
import jax
import jax.numpy as jnp
from jax.experimental import pallas as pl

_N_ITERS = 10
_BLOCK_ROWS = 512


def _sparsemax_block(x_ref, o_ref):
    x = x_ref[...]
    mx = jnp.max(x, axis=1, keepdims=True)
    lo = mx - 1.0
    hi = mx

    def body(_, carry):
        lo, hi = carry
        mid = 0.5 * (lo + hi)
        f = jnp.sum(jnp.maximum(x - mid, 0.0), axis=1, keepdims=True)
        gt = f > 1.0
        lo = jnp.where(gt, mid, lo)
        hi = jnp.where(gt, hi, mid)
        return lo, hi

    lo, hi = jax.lax.fori_loop(0, _N_ITERS, body, (lo, hi))
    mask = x > lo
    s = jnp.sum(jnp.where(mask, x, 0.0), axis=1, keepdims=True)
    k = jnp.sum(mask.astype(x.dtype), axis=1, keepdims=True)
    tau = jnp.clip((s - 1.0) / k, lo, hi)
    o_ref[...] = jnp.maximum(x - tau, 0.0)


def kernel(input):
    orig_shape = input.shape
    n = orig_shape[-1]
    x2 = input.reshape(-1, n)
    rows = x2.shape[0]
    out = pl.pallas_call(
        _sparsemax_block,
        grid=(rows // _BLOCK_ROWS,),
        in_specs=[pl.BlockSpec((_BLOCK_ROWS, n), lambda i: (i, 0))],
        out_specs=pl.BlockSpec((_BLOCK_ROWS, n), lambda i: (i, 0)),
        out_shape=jax.ShapeDtypeStruct((rows, n), x2.dtype),
    )(x2)
    return out.reshape(orig_shape)


# speedup vs baseline: 2.0444x; 1.0194x over previous
import jax
import jax.numpy as jnp
from jax.experimental import pallas as pl

_N_ITERS = 10
_BLOCK_ROWS = 1024


def _sparsemax_block(x_ref, o_ref):
    x = x_ref[...]
    mx = jnp.max(x, axis=1, keepdims=True)
    lo = mx - 1.0
    hi = mx

    def body(_, carry):
        lo, hi = carry
        mid = 0.5 * (lo + hi)
        f = jnp.sum(jnp.maximum(x - mid, 0.0), axis=1, keepdims=True)
        gt = f > 1.0
        lo = jnp.where(gt, mid, lo)
        hi = jnp.where(gt, hi, mid)
        return lo, hi

    lo, hi = jax.lax.fori_loop(0, _N_ITERS, body, (lo, hi))
    mask = x > lo
    s = jnp.sum(jnp.where(mask, x, 0.0), axis=1, keepdims=True)
    k = jnp.sum(mask.astype(x.dtype), axis=1, keepdims=True)
    tau = jnp.clip((s - 1.0) / k, lo, hi)
    o_ref[...] = jnp.maximum(x - tau, 0.0)


def kernel(input):
    orig_shape = input.shape
    n = orig_shape[-1]
    x2 = input.reshape(-1, n)
    rows = x2.shape[0]
    out = pl.pallas_call(
        _sparsemax_block,
        grid=(rows // _BLOCK_ROWS,),
        in_specs=[pl.BlockSpec((_BLOCK_ROWS, n), lambda i: (i, 0))],
        out_specs=pl.BlockSpec((_BLOCK_ROWS, n), lambda i: (i, 0)),
        out_shape=jax.ShapeDtypeStruct((rows, n), x2.dtype),
    )(x2)
    return out.reshape(orig_shape)


# speedup vs baseline: 2.2959x; 1.1230x over previous
import jax
import jax.numpy as jnp
from jax.experimental import pallas as pl

_N_ITERS_FAST = 7
_N_ITERS_EXACT = 3
_BLOCK_ROWS = 1024


def _sparsemax_block(x_ref, o_ref):
    x = x_ref[...]
    n = x.shape[1]
    mx = jnp.max(x, axis=1, keepdims=True)
    lo = mx - 1.0
    hi = mx

    def body_fast(_, carry):
        lo, hi = carry
        mid = 0.5 * (lo + hi)
        sm = jnp.sum(jnp.maximum(x, mid), axis=1, keepdims=True)
        gt = sm > 1.0 + n * mid
        lo = jnp.where(gt, mid, lo)
        hi = jnp.where(gt, hi, mid)
        return lo, hi

    def body_exact(_, carry):
        lo, hi = carry
        mid = 0.5 * (lo + hi)
        f = jnp.sum(jnp.maximum(x - mid, 0.0), axis=1, keepdims=True)
        gt = f > 1.0
        lo = jnp.where(gt, mid, lo)
        hi = jnp.where(gt, hi, mid)
        return lo, hi

    lo, hi = jax.lax.fori_loop(0, _N_ITERS_FAST, body_fast, (lo, hi))
    lo, hi = jax.lax.fori_loop(0, _N_ITERS_EXACT, body_exact, (lo, hi))
    mask = x > lo
    s = jnp.sum(jnp.where(mask, x, 0.0), axis=1, keepdims=True)
    k = jnp.sum(mask.astype(x.dtype), axis=1, keepdims=True)
    tau = jnp.clip((s - 1.0) / k, lo, hi)
    o_ref[...] = jnp.maximum(x - tau, 0.0)


def kernel(input):
    orig_shape = input.shape
    n = orig_shape[-1]
    x2 = input.reshape(-1, n)
    rows = x2.shape[0]
    out = pl.pallas_call(
        _sparsemax_block,
        grid=(rows // _BLOCK_ROWS,),
        in_specs=[pl.BlockSpec((_BLOCK_ROWS, n), lambda i: (i, 0))],
        out_specs=pl.BlockSpec((_BLOCK_ROWS, n), lambda i: (i, 0)),
        out_shape=jax.ShapeDtypeStruct((rows, n), x2.dtype),
    )(x2)
    return out.reshape(orig_shape)
